# Initial kernel scaffold; baseline (speedup 1.0000x reference)
import jax
import jax.numpy as jnp
from jax.experimental import pallas as pl


def kernel(mlmm_R, ml_idxi, ml_idxj, ml_Z, emb, centers, widths, Wrbf, Wi, bi, Wj, bj, Wri1, bri1, Wri2, bri2, u, Wmi, bmi, Wra1, bra1, Wra2, bra2, Wro1, bro1, Wro2, bro2, Wout, bout, Escale, Eshift, Qscale, Qshift):
    raise NotImplementedError("write your pallas kernel here")


# speedup vs baseline: 3.0934x; 3.0934x over previous
import functools

import jax
import jax.numpy as jnp
from jax import lax
from jax.experimental import pallas as pl
from jax.experimental.pallas import tpu as pltpu
from jax.experimental.pallas import tpu_sc as plsc

F = 128
K = 64
NB = 3
NRI = 2
NRA = 2
NRO = 1
N = 10000
E = 320000
ML_CUT = 10.0

NC = 2
NS = 16
NW = NC * NS
EP = E // NW
C = 80
CPT = EP // C
RPS = N // NS

_LN2 = 0.6931471805599453


def _ssp(x):
    return jnp.logaddexp(x, 0.0) - _LN2


def _geom_body(R_hbm, ii_hbm, jj_hbm, cf_hbm, ed_hbm,
               R_v, ii_v, jj_v, cf_v, ed_v):
    cid = lax.axis_index("c")
    sid = lax.axis_index("s")
    w = sid * NC + cid
    base = w * EP

    pltpu.sync_copy(R_hbm, R_v)
    pltpu.sync_copy(ii_hbm.at[pl.ds(base, EP)], ii_v)
    pltpu.sync_copy(jj_hbm.at[pl.ds(base, EP)], jj_v)

    def step(t, carry):
        o = t * 16
        ii = ii_v[pl.ds(o, 16)] * 3
        jj = jj_v[pl.ds(o, 16)] * 3
        ax = plsc.load_gather(R_v, [ii])
        ay = plsc.load_gather(R_v, [ii + 1])
        az = plsc.load_gather(R_v, [ii + 2])
        bx = plsc.load_gather(R_v, [jj])
        by = plsc.load_gather(R_v, [jj + 1])
        bz = plsc.load_gather(R_v, [jj + 2])
        dx = ax - bx
        dy = ay - by
        dz = az - bz
        dsq = jnp.maximum(dx * dx + dy * dy + dz * dz, 1e-12)
        bits = plsc.bitcast(dsq, jnp.int32)
        y = plsc.bitcast(jnp.int32(0x5F3759DF) - (bits >> 1), jnp.float32)
        for _ in range(3):
            y = y * (1.5 - 0.5 * dsq * y * y)
        dist = dsq * y
        x = dist * (1.0 / ML_CUT)
        x3 = x * x * x
        x4 = x3 * x
        x5 = x4 * x
        cf = jnp.where(x < 1.0, 1.0 - 6.0 * x5 + 15.0 * x4 - 10.0 * x3, 0.0)
        cf_v[pl.ds(o, 16)] = cf
        ed_v[pl.ds(o, 16)] = jnp.exp(-dist)
        return carry

    lax.fori_loop(0, EP // 16, step, 0)

    pltpu.sync_copy(cf_v, cf_hbm.at[pl.ds(base, EP)])
    pltpu.sync_copy(ed_v, ed_hbm.at[pl.ds(base, EP)])


def _make_geom():
    mesh = plsc.VectorSubcoreMesh(core_axis_name="c", subcore_axis_name="s")
    return pl.kernel(
        _geom_body,
        compiler_params=pltpu.CompilerParams(needs_layout_passes=False, use_tc_tiling_on_sc=False),
        out_type=[jax.ShapeDtypeStruct((E,), jnp.float32),
                  jax.ShapeDtypeStruct((E,), jnp.float32)],
        mesh=mesh,
        scratch_types=[
            pltpu.VMEM((N * 3,), jnp.float32),
            pltpu.VMEM((EP,), jnp.int32),
            pltpu.VMEM((EP,), jnp.int32),
            pltpu.VMEM((EP,), jnp.float32),
            pltpu.VMEM((EP,), jnp.float32),
        ],
    )


def _agg_body(xj_hbm, g_hbm, ii2_hbm, jj2_hbm, out_hbm,
              ii_v, jj_v, rows_v, g_v, macc_sh, sem):
    cid = lax.axis_index("c")
    sid = lax.axis_index("s")
    w = sid * NC + cid

    zeros16 = jnp.zeros((16,), jnp.float32)

    def zstep(r, carry):
        for j in range(8):
            rows_v[r, pl.ds(j * 16, 16)] = zeros16
        return carry

    lax.fori_loop(0, C, zstep, 0)
    rbase = sid * RPS
    for k in range(7):
        pltpu.sync_copy(rows_v, macc_sh.at[pl.ds(rbase + k * C, C)])
    pltpu.sync_copy(rows_v.at[pl.ds(0, RPS - 7 * C)],
                    macc_sh.at[pl.ds(rbase + 7 * C, RPS - 7 * C)])

    pltpu.sync_copy(ii2_hbm.at[pl.ds(w * CPT, CPT)], ii_v)
    pltpu.sync_copy(jj2_hbm.at[pl.ds(w * CPT, CPT)], jj_v)

    plsc.subcore_barrier()

    def chunk(t, carry):
        ebase = w * EP + t * C
        pltpu.async_copy(xj_hbm.at[jj_v.at[t]], rows_v, sem).wait()
        pltpu.sync_copy(g_hbm.at[pl.ds(ebase, C)], g_v)

        def mstep(r, carry2):
            for j in range(8):
                s = pl.ds(j * 16, 16)
                g_v[r, s] = g_v[r, s] * rows_v[r, s]
            return carry2

        lax.fori_loop(0, C, mstep, 0)
        pltpu.sync_copy(g_v, macc_sh.at[ii_v.at[t]], add=True)
        return carry

    lax.fori_loop(0, CPT, chunk, 0)

    plsc.subcore_barrier()
    pltpu.sync_copy(macc_sh.at[pl.ds(rbase, RPS)],
                    out_hbm.at[cid, pl.ds(rbase, RPS)])


def _make_agg():
    mesh = plsc.VectorSubcoreMesh(core_axis_name="c", subcore_axis_name="s")
    return pl.kernel(
        _agg_body,
        compiler_params=pltpu.CompilerParams(needs_layout_passes=False, use_tc_tiling_on_sc=False),
        out_type=jax.ShapeDtypeStruct((NC, N, F), jnp.float32),
        mesh=mesh,
        scratch_types=[
            pltpu.VMEM((CPT, C), jnp.int32),
            pltpu.VMEM((CPT, C), jnp.int32),
            pltpu.VMEM((C, F), jnp.float32),
            pltpu.VMEM((C, F), jnp.float32),
            pltpu.VMEM_SHARED((N, F), jnp.float32),
            pltpu.SemaphoreType.DMA,
        ],
    )


TE = 1280
TN = 1000


def _g_kernel(cf_ref, ed_ref, cen_ref, wid_ref, W_ref, out_ref):
    cf = cf_ref[:]
    ed = ed_ref[:]
    cen = cen_ref[:]
    wid = wid_ref[:]
    t = ed - cen
    rbf = cf * jnp.exp(-wid * t * t)
    out_ref[:] = jnp.dot(rbf, W_ref[:], preferred_element_type=jnp.float32)


def _run_g(cf, ed, centers, widths, Wrbf_b):
    cf2 = cf.reshape(E, 1)
    ed2 = ed.reshape(E, 1)
    return pl.pallas_call(
        _g_kernel,
        grid=(E // TE,),
        in_specs=[
            pl.BlockSpec((TE, 1), lambda i: (i, 0)),
            pl.BlockSpec((TE, 1), lambda i: (i, 0)),
            pl.BlockSpec((1, K), lambda i: (0, 0)),
            pl.BlockSpec((1, K), lambda i: (0, 0)),
            pl.BlockSpec((K, F), lambda i: (0, 0)),
        ],
        out_specs=pl.BlockSpec((TE, F), lambda i: (i, 0)),
        out_shape=jax.ShapeDtypeStruct((E, F), jnp.float32),
    )(cf2, ed2, centers.reshape(1, K), widths.reshape(1, K), Wrbf_b)


def _b0_kernel(z_ref, emb_ref, Wi_ref, bi_ref, Wj_ref, bj_ref,
               x0_ref, xi_ref, xj_ref):
    z = z_ref[...].reshape(TN)
    oh = (lax.broadcasted_iota(jnp.int32, (TN, 95), 1) == z[:, None]
          ).astype(jnp.float32)
    x = jnp.dot(oh, emb_ref[:], preferred_element_type=jnp.float32)
    x0_ref[:] = x
    xa = _ssp(x)
    xi_ref[:] = _ssp(jnp.dot(xa, Wi_ref[:], preferred_element_type=jnp.float32)
                     + bi_ref[:])
    xj_ref[:] = _ssp(jnp.dot(xa, Wj_ref[:], preferred_element_type=jnp.float32)
                     + bj_ref[:])


def _run_b0(ml_Z, emb, Wi_b, bi_b, Wj_b, bj_b):
    return pl.pallas_call(
        _b0_kernel,
        grid=(N // TN,),
        in_specs=[
            pl.BlockSpec((1, 1, TN), lambda i: (i, 0, 0)),
            pl.BlockSpec((95, F), lambda i: (0, 0)),
            pl.BlockSpec((F, F), lambda i: (0, 0)),
            pl.BlockSpec((F,), lambda i: (0,)),
            pl.BlockSpec((F, F), lambda i: (0, 0)),
            pl.BlockSpec((F,), lambda i: (0,)),
        ],
        out_specs=[
            pl.BlockSpec((TN, F), lambda i: (i, 0)),
            pl.BlockSpec((TN, F), lambda i: (i, 0)),
            pl.BlockSpec((TN, F), lambda i: (i, 0)),
        ],
        out_shape=[
            jax.ShapeDtypeStruct((N, F), jnp.float32),
            jax.ShapeDtypeStruct((N, F), jnp.float32),
            jax.ShapeDtypeStruct((N, F), jnp.float32),
        ],
    )(ml_Z, emb, Wi_b, bi_b, Wj_b, bj_b)


def _b_kernel(x_ref, Wi_ref, bi_ref, Wj_ref, bj_ref, xi_ref, xj_ref):
    xa = _ssp(x_ref[:])
    xi_ref[:] = _ssp(jnp.dot(xa, Wi_ref[:], preferred_element_type=jnp.float32)
                     + bi_ref[:])
    xj_ref[:] = _ssp(jnp.dot(xa, Wj_ref[:], preferred_element_type=jnp.float32)
                     + bj_ref[:])


def _run_b(x, Wi_b, bi_b, Wj_b, bj_b):
    return pl.pallas_call(
        _b_kernel,
        grid=(N // TN,),
        in_specs=[
            pl.BlockSpec((TN, F), lambda i: (i, 0)),
            pl.BlockSpec((F, F), lambda i: (0, 0)),
            pl.BlockSpec((F,), lambda i: (0,)),
            pl.BlockSpec((F, F), lambda i: (0, 0)),
            pl.BlockSpec((F,), lambda i: (0,)),
        ],
        out_specs=[
            pl.BlockSpec((TN, F), lambda i: (i, 0)),
            pl.BlockSpec((TN, F), lambda i: (i, 0)),
        ],
        out_shape=[
            jax.ShapeDtypeStruct((N, F), jnp.float32),
            jax.ShapeDtypeStruct((N, F), jnp.float32),
        ],
    )(x, Wi_b, bi_b, Wj_b, bj_b)


def _matmul(a, w_ref, b_ref=None):
    out = jnp.dot(a, w_ref[:], preferred_element_type=jnp.float32)
    if b_ref is not None:
        out = out + b_ref[:]
    return out


def _c_kernel(final, x_ref, xi_ref, p_ref, eacc_ref,
              Wri1_ref, bri1_ref, Wri2_ref, bri2_ref, u_ref,
              Wmi_ref, bmi_ref, Wra1_ref, bra1_ref, Wra2_ref, bra2_ref,
              Wro1_ref, bro1_ref, Wro2_ref, bro2_ref,
              Wout_ref, bout_ref, z_ref, scale_ref, shift_ref,
              xout_ref, eout_ref):
    m = xi_ref[:] + p_ref[0] + p_ref[1]
    for r in range(NRI):
        v = _ssp(m)
        v = _ssp(_matmul(v, Wri1_ref.at[r], bri1_ref.at[r]))
        v = _matmul(v, Wri2_ref.at[r], bri2_ref.at[r])
        m = m + v
    x = u_ref[:] * x_ref[:] + _matmul(_ssp(m), Wmi_ref, bmi_ref)
    for r in range(NRA):
        v = _ssp(x)
        v = _ssp(_matmul(v, Wra1_ref.at[r], bra1_ref.at[r]))
        v = _matmul(v, Wra2_ref.at[r], bra2_ref.at[r])
        x = x + v
    xout_ref[:] = x
    y = x
    for r in range(NRO):
        v = _ssp(y)
        v = _ssp(_matmul(v, Wro1_ref.at[r], bro1_ref.at[r]))
        v = _matmul(v, Wro2_ref.at[r], bro2_ref.at[r])
        y = y + v
    eacc = eacc_ref[:] + _matmul(_ssp(y), Wout_ref, bout_ref)
    if final:
        z = z_ref[...].reshape(TN)
        oh = (lax.broadcasted_iota(jnp.int32, (TN, 95), 1) == z[:, None]
              ).astype(jnp.float32)
        sc = jnp.dot(oh, scale_ref[:], preferred_element_type=jnp.float32)
        sh = jnp.dot(oh, shift_ref[:], preferred_element_type=jnp.float32)
        eacc = sc * eacc + sh
    eout_ref[:] = eacc


def _run_c(final, x, xi, p, eacc, Wri1_b, bri1_b, Wri2_b, bri2_b, u_b,
           Wmi_b, bmi_b, Wra1_b, bra1_b, Wra2_b, bra2_b,
           Wro1_b, bro1_b, Wro2_b, bro2_b, Woutp, boutp, ml_Z, scalep, shiftp):
    full = lambda *s: pl.BlockSpec(s, lambda i: (0,) * len(s))
    row = lambda *s: pl.BlockSpec((s[0],) + s[1:],
                                  lambda i: (i,) + (0,) * (len(s) - 1))
    return pl.pallas_call(
        functools.partial(_c_kernel, final),
        grid=(N // TN,),
        in_specs=[
            row(TN, F), row(TN, F),
            pl.BlockSpec((NC, TN, F), lambda i: (0, i, 0)),
            row(TN, F),
            full(NRI, F, F), full(NRI, F), full(NRI, F, F), full(NRI, F),
            full(F), full(F, F), full(F),
            full(NRA, F, F), full(NRA, F), full(NRA, F, F), full(NRA, F),
            full(NRO, F, F), full(NRO, F), full(NRO, F, F), full(NRO, F),
            full(F, F), full(F),
            pl.BlockSpec((1, 1, TN), lambda i: (i, 0, 0)),
            full(95, F), full(95, F),
        ],
        out_specs=[row(TN, F), row(TN, F)],
        out_shape=[
            jax.ShapeDtypeStruct((N, F), jnp.float32),
            jax.ShapeDtypeStruct((N, F), jnp.float32),
        ],
    )(x, xi, p, eacc, Wri1_b, bri1_b, Wri2_b, bri2_b, u_b, Wmi_b, bmi_b,
      Wra1_b, bra1_b, Wra2_b, bra2_b, Wro1_b, bro1_b, Wro2_b, bro2_b,
      Woutp, boutp, ml_Z, scalep, shiftp)


def kernel(mlmm_R, ml_idxi, ml_idxj, ml_Z, emb, centers, widths, Wrbf,
           Wi, bi, Wj, bj, Wri1, bri1, Wri2, bri2, u, Wmi, bmi,
           Wra1, bra1, Wra2, bra2, Wro1, bro1, Wro2, bro2, Wout, bout,
           Escale, Eshift, Qscale, Qshift):
    idxi = ml_idxi.astype(jnp.int32)
    idxj = ml_idxj.astype(jnp.int32)
    z32 = ml_Z.astype(jnp.int32)
    z3 = z32.reshape(N // TN, 1, TN)

    cf, ed = _make_geom()(mlmm_R.reshape(N * 3), idxi, idxj)

    ii2 = idxi.reshape(E // C, C)
    jj2 = idxj.reshape(E // C, C)

    Woutp = jnp.zeros((NB, F, F), jnp.float32).at[:, :, :2].set(Wout)
    boutp = jnp.zeros((NB, F), jnp.float32).at[:, :2].set(bout)
    scalep = jnp.zeros((95, F), jnp.float32).at[:, 0].set(Escale)
    scalep = scalep.at[:, 1].set(Qscale)
    shiftp = jnp.zeros((95, F), jnp.float32).at[:, 0].set(Eshift)
    shiftp = shiftp.at[:, 1].set(Qshift)

    eacc = jnp.zeros((N, F), jnp.float32)
    agg = _make_agg()
    x = None
    for b in range(NB):
        if b == 0:
            x, xi, xj = _run_b0(z3, emb, Wi[0], bi[0], Wj[0], bj[0])
        else:
            xi, xj = _run_b(x, Wi[b], bi[b], Wj[b], bj[b])
        g = _run_g(cf, ed, centers, widths, Wrbf[b])
        p = agg(xj, g, ii2, jj2)
        x, eacc = _run_c(b == NB - 1, x, xi, p, eacc,
                         Wri1[b], bri1[b], Wri2[b], bri2[b], u[b],
                         Wmi[b], bmi[b], Wra1[b], bra1[b], Wra2[b], bra2[b],
                         Wro1[b], bro1[b], Wro2[b], bro2[b],
                         Woutp[b], boutp[b], z3, scalep, shiftp)
    return eacc[:, :2]


# speedup vs baseline: 3.9274x; 1.2696x over previous
import functools

import jax
import jax.numpy as jnp
from jax import lax
from jax.experimental import pallas as pl
from jax.experimental.pallas import tpu as pltpu
from jax.experimental.pallas import tpu_sc as plsc

F = 128
K = 64
NB = 3
NRI = 2
NRA = 2
NRO = 1
N = 10000
E = 320000
ML_CUT = 10.0

NC = 2
NS = 16
NW = NC * NS
EP = E // NW
C = 80
CPT = EP // C
RPS = N // NS

_LN2 = 0.6931471805599453


def _ssp(x):
    return jnp.logaddexp(x, 0.0) - _LN2


def _geom_body(R_hbm, ii_hbm, jj_hbm, cf_hbm, ed_hbm,
               R_v, ii_v, jj_v, cf_v, ed_v):
    cid = lax.axis_index("c")
    sid = lax.axis_index("s")
    w = sid * NC + cid
    base = w * EP

    pltpu.sync_copy(R_hbm, R_v)
    pltpu.sync_copy(ii_hbm.at[pl.ds(base, EP)], ii_v)
    pltpu.sync_copy(jj_hbm.at[pl.ds(base, EP)], jj_v)

    def step(t, carry):
        o = t * 16
        ii = ii_v[pl.ds(o, 16)] * 3
        jj = jj_v[pl.ds(o, 16)] * 3
        ax = plsc.load_gather(R_v, [ii])
        ay = plsc.load_gather(R_v, [ii + 1])
        az = plsc.load_gather(R_v, [ii + 2])
        bx = plsc.load_gather(R_v, [jj])
        by = plsc.load_gather(R_v, [jj + 1])
        bz = plsc.load_gather(R_v, [jj + 2])
        dx = ax - bx
        dy = ay - by
        dz = az - bz
        dsq = jnp.maximum(dx * dx + dy * dy + dz * dz, 1e-12)
        bits = plsc.bitcast(dsq, jnp.int32)
        y = plsc.bitcast(jnp.int32(0x5F3759DF) - (bits >> 1), jnp.float32)
        for _ in range(3):
            y = y * (1.5 - 0.5 * dsq * y * y)
        dist = dsq * y
        x = dist * (1.0 / ML_CUT)
        x3 = x * x * x
        x4 = x3 * x
        x5 = x4 * x
        cf = jnp.where(x < 1.0, 1.0 - 6.0 * x5 + 15.0 * x4 - 10.0 * x3, 0.0)
        cf_v[pl.ds(o, 16)] = cf
        ed_v[pl.ds(o, 16)] = dist
        return carry

    lax.fori_loop(0, EP // 16, step, 0)

    pltpu.sync_copy(cf_v, cf_hbm.at[pl.ds(base, EP)])
    pltpu.sync_copy(ed_v, ed_hbm.at[pl.ds(base, EP)])


def _make_geom():
    mesh = plsc.VectorSubcoreMesh(core_axis_name="c", subcore_axis_name="s")
    return pl.kernel(
        _geom_body,
        compiler_params=pltpu.CompilerParams(needs_layout_passes=False, use_tc_tiling_on_sc=False),
        out_type=[jax.ShapeDtypeStruct((E,), jnp.float32),
                  jax.ShapeDtypeStruct((E,), jnp.float32)],
        mesh=mesh,
        scratch_types=[
            pltpu.VMEM((N * 3,), jnp.float32),
            pltpu.VMEM((EP,), jnp.int32),
            pltpu.VMEM((EP,), jnp.int32),
            pltpu.VMEM((EP,), jnp.float32),
            pltpu.VMEM((EP,), jnp.float32),
        ],
    )


SLAB = 25
NSLAB = CPT // SLAB


def _agg_body(xj_hbm, g_hbm, ii2_hbm, jj2_hbm, out_hbm,
              ii_v, jj_v, rv, gv, macc_sh,
              xsem0, xsem1, gsem0, gsem1, ssem0, ssem1, isem):
    xsem = (xsem0, xsem1)
    gsem = (gsem0, gsem1)
    ssem = (ssem0, ssem1)
    cid = lax.axis_index("c")
    sid = lax.axis_index("s")
    w = sid * NC + cid

    zeros16 = jnp.zeros((16,), jnp.float32)

    def zstep(r, carry):
        for j in range(8):
            rv[r, pl.ds(j * 16, 16)] = zeros16
        return carry

    lax.fori_loop(0, C, zstep, 0)
    rbase = sid * RPS
    zb = rv.at[pl.ds(0, C)]
    for k in range(7):
        pltpu.sync_copy(zb, macc_sh.at[pl.ds(rbase + k * C, C)])
    pltpu.sync_copy(rv.at[pl.ds(0, RPS - 7 * C)],
                    macc_sh.at[pl.ds(rbase + 7 * C, RPS - 7 * C)])

    pltpu.sync_copy(ii2_hbm.at[pl.ds(w * CPT, SLAB)], ii_v.at[pl.ds(0, SLAB)])
    pltpu.sync_copy(jj2_hbm.at[pl.ds(w * CPT, SLAB)], jj_v.at[pl.ds(0, SLAB)])

    plsc.subcore_barrier()

    def idxrow(ring, c):
        return ring.at[lax.rem(c // SLAB, 2) * SLAB + lax.rem(c, SLAB)]

    def xissue(c, j):
        pltpu.async_copy(xj_hbm.at[idxrow(jj_v, c)],
                         rv.at[pl.ds(j * C, C)], xsem[j])

    def xwait(j):
        pltpu.make_async_copy(xj_hbm.at[jj_v.at[0]],
                              rv.at[pl.ds(j * C, C)], xsem[j]).wait()

    def gissue(c, j):
        pltpu.async_copy(g_hbm.at[pl.ds(w * EP + c * C, C)],
                         gv.at[pl.ds(j * C, C)], gsem[j])

    def gwait(j):
        pltpu.make_async_copy(g_hbm.at[pl.ds(0, C)],
                              gv.at[pl.ds(j * C, C)], gsem[j]).wait()

    def sissue(t, j):
        pltpu.async_copy(gv.at[pl.ds(j * C, C)],
                         macc_sh.at[idxrow(ii_v, t)], ssem[j], add=True)

    def swait(j):
        pltpu.make_async_copy(gv.at[pl.ds(j * C, C)],
                              macc_sh.at[ii_v.at[0]], ssem[j]).wait()

    def mul(j):
        def mstep(r, carry):
            for u_ in range(8):
                s = pl.ds(u_ * 16, 16)
                gv[j * C + r, s] = gv[j * C + r, s] * rv[j * C + r, s]
            return carry

        lax.fori_loop(0, C, mstep, 0)

    xissue(0, 0)
    xissue(1, 1)
    gissue(0, 0)

    @pl.loop(0, (CPT - 1) // 2)
    def _(q):
        for j in range(2):
            t = 2 * q + j
            jp = 1 - j

            @pl.when((lax.rem(t, SLAB) == 5) & (t < (NSLAB - 1) * SLAB))
            def _():
                s_next = t // SLAB + 1
                dst = lax.rem(s_next, 2) * SLAB
                pltpu.async_copy(ii2_hbm.at[pl.ds(w * CPT + s_next * SLAB,
                                                  SLAB)],
                                 ii_v.at[pl.ds(dst, SLAB)], isem)
                pltpu.async_copy(jj2_hbm.at[pl.ds(w * CPT + s_next * SLAB,
                                                  SLAB)],
                                 jj_v.at[pl.ds(dst, SLAB)], isem)

            @pl.when((lax.rem(t, SLAB) == SLAB - 2)
                     & (t < (NSLAB - 1) * SLAB))
            def _():
                pltpu.make_async_copy(ii2_hbm.at[pl.ds(0, SLAB)],
                                      ii_v.at[pl.ds(0, SLAB)], isem).wait()
                pltpu.make_async_copy(jj2_hbm.at[pl.ds(0, SLAB)],
                                      jj_v.at[pl.ds(0, SLAB)], isem).wait()

            xwait(j)
            gwait(j)
            mul(j)

            if j == 0:
                @pl.when(q < (CPT - 1) // 2)
                def _():
                    xissue(t + 2, j)
            else:
                @pl.when(q < (CPT - 1) // 2 - 1)
                def _():
                    xissue(t + 2, j)

            if j == 0:
                @pl.when(q > 0)
                def _():
                    swait(jp)
            else:
                swait(jp)
            gissue(t + 1, jp)

            sissue(t, j)

    xwait(0)
    gwait(0)
    mul(0)
    swait(1)
    sissue(CPT - 1, 0)
    swait(0)

    plsc.subcore_barrier()
    pltpu.sync_copy(macc_sh.at[pl.ds(rbase, RPS)],
                    out_hbm.at[cid, pl.ds(rbase, RPS)])


def _make_agg():
    mesh = plsc.VectorSubcoreMesh(core_axis_name="c", subcore_axis_name="s")
    return pl.kernel(
        _agg_body,
        compiler_params=pltpu.CompilerParams(needs_layout_passes=False, use_tc_tiling_on_sc=False),
        out_type=jax.ShapeDtypeStruct((NC, N, F), jnp.float32),
        mesh=mesh,
        scratch_types=[
            pltpu.VMEM((2 * SLAB, C), jnp.int32),
            pltpu.VMEM((2 * SLAB, C), jnp.int32),
            pltpu.VMEM((2 * C, F), jnp.float32),
            pltpu.VMEM((2 * C, F), jnp.float32),
            pltpu.VMEM_SHARED((N, F), jnp.float32),
        ] + [pltpu.SemaphoreType.DMA] * 7,
    )


TE = 1280
TN = 1000


def _g_kernel(cf_ref, ed_ref, cen_ref, wid_ref, W_ref, out_ref):
    cf = cf_ref[:]
    ed = jnp.exp(-ed_ref[:])
    cen = cen_ref[:]
    wid = wid_ref[:]
    t = ed - cen
    rbf = cf * jnp.exp(-wid * t * t)
    out_ref[:] = jnp.dot(rbf, W_ref[:], preferred_element_type=jnp.float32)


def _run_g(cf, ed, centers, widths, Wrbf_b):
    cf2 = cf.reshape(E, 1)
    ed2 = ed.reshape(E, 1)
    return pl.pallas_call(
        _g_kernel,
        grid=(E // TE,),
        in_specs=[
            pl.BlockSpec((TE, 1), lambda i: (i, 0)),
            pl.BlockSpec((TE, 1), lambda i: (i, 0)),
            pl.BlockSpec((1, K), lambda i: (0, 0)),
            pl.BlockSpec((1, K), lambda i: (0, 0)),
            pl.BlockSpec((K, F), lambda i: (0, 0)),
        ],
        out_specs=pl.BlockSpec((TE, F), lambda i: (i, 0)),
        out_shape=jax.ShapeDtypeStruct((E, F), jnp.float32),
    )(cf2, ed2, centers.reshape(1, K), widths.reshape(1, K), Wrbf_b)


def _b0_kernel(z_ref, emb_ref, Wi_ref, bi_ref, Wj_ref, bj_ref,
               x0_ref, xi_ref, xj_ref):
    z = z_ref[...].reshape(TN)
    oh = (lax.broadcasted_iota(jnp.int32, (TN, 95), 1) == z[:, None]
          ).astype(jnp.float32)
    x = jnp.dot(oh, emb_ref[:], preferred_element_type=jnp.float32)
    x0_ref[:] = x
    xa = _ssp(x)
    xi_ref[:] = _ssp(jnp.dot(xa, Wi_ref[:], preferred_element_type=jnp.float32)
                     + bi_ref[:])
    xj_ref[:] = _ssp(jnp.dot(xa, Wj_ref[:], preferred_element_type=jnp.float32)
                     + bj_ref[:])


def _run_b0(ml_Z, emb, Wi_b, bi_b, Wj_b, bj_b):
    return pl.pallas_call(
        _b0_kernel,
        grid=(N // TN,),
        in_specs=[
            pl.BlockSpec((1, 1, TN), lambda i: (i, 0, 0)),
            pl.BlockSpec((95, F), lambda i: (0, 0)),
            pl.BlockSpec((F, F), lambda i: (0, 0)),
            pl.BlockSpec((F,), lambda i: (0,)),
            pl.BlockSpec((F, F), lambda i: (0, 0)),
            pl.BlockSpec((F,), lambda i: (0,)),
        ],
        out_specs=[
            pl.BlockSpec((TN, F), lambda i: (i, 0)),
            pl.BlockSpec((TN, F), lambda i: (i, 0)),
            pl.BlockSpec((TN, F), lambda i: (i, 0)),
        ],
        out_shape=[
            jax.ShapeDtypeStruct((N, F), jnp.float32),
            jax.ShapeDtypeStruct((N, F), jnp.float32),
            jax.ShapeDtypeStruct((N, F), jnp.float32),
        ],
    )(ml_Z, emb, Wi_b, bi_b, Wj_b, bj_b)


def _b_kernel(x_ref, Wi_ref, bi_ref, Wj_ref, bj_ref, xi_ref, xj_ref):
    xa = _ssp(x_ref[:])
    xi_ref[:] = _ssp(jnp.dot(xa, Wi_ref[:], preferred_element_type=jnp.float32)
                     + bi_ref[:])
    xj_ref[:] = _ssp(jnp.dot(xa, Wj_ref[:], preferred_element_type=jnp.float32)
                     + bj_ref[:])


def _run_b(x, Wi_b, bi_b, Wj_b, bj_b):
    return pl.pallas_call(
        _b_kernel,
        grid=(N // TN,),
        in_specs=[
            pl.BlockSpec((TN, F), lambda i: (i, 0)),
            pl.BlockSpec((F, F), lambda i: (0, 0)),
            pl.BlockSpec((F,), lambda i: (0,)),
            pl.BlockSpec((F, F), lambda i: (0, 0)),
            pl.BlockSpec((F,), lambda i: (0,)),
        ],
        out_specs=[
            pl.BlockSpec((TN, F), lambda i: (i, 0)),
            pl.BlockSpec((TN, F), lambda i: (i, 0)),
        ],
        out_shape=[
            jax.ShapeDtypeStruct((N, F), jnp.float32),
            jax.ShapeDtypeStruct((N, F), jnp.float32),
        ],
    )(x, Wi_b, bi_b, Wj_b, bj_b)


def _matmul(a, w_ref, b_ref=None):
    out = jnp.dot(a, w_ref[:], preferred_element_type=jnp.float32)
    if b_ref is not None:
        out = out + b_ref[:]
    return out


def _c_kernel(final, x_ref, xi_ref, p_ref, eacc_ref,
              Wri1_ref, bri1_ref, Wri2_ref, bri2_ref, u_ref,
              Wmi_ref, bmi_ref, Wra1_ref, bra1_ref, Wra2_ref, bra2_ref,
              Wro1_ref, bro1_ref, Wro2_ref, bro2_ref,
              Wout_ref, bout_ref, z_ref, scale_ref, shift_ref,
              xout_ref, eout_ref):
    m = xi_ref[:] + p_ref[0] + p_ref[1]
    for r in range(NRI):
        v = _ssp(m)
        v = _ssp(_matmul(v, Wri1_ref.at[r], bri1_ref.at[r]))
        v = _matmul(v, Wri2_ref.at[r], bri2_ref.at[r])
        m = m + v
    x = u_ref[:] * x_ref[:] + _matmul(_ssp(m), Wmi_ref, bmi_ref)
    for r in range(NRA):
        v = _ssp(x)
        v = _ssp(_matmul(v, Wra1_ref.at[r], bra1_ref.at[r]))
        v = _matmul(v, Wra2_ref.at[r], bra2_ref.at[r])
        x = x + v
    xout_ref[:] = x
    y = x
    for r in range(NRO):
        v = _ssp(y)
        v = _ssp(_matmul(v, Wro1_ref.at[r], bro1_ref.at[r]))
        v = _matmul(v, Wro2_ref.at[r], bro2_ref.at[r])
        y = y + v
    eacc = eacc_ref[:] + _matmul(_ssp(y), Wout_ref, bout_ref)
    if final:
        z = z_ref[...].reshape(TN)
        oh = (lax.broadcasted_iota(jnp.int32, (TN, 95), 1) == z[:, None]
              ).astype(jnp.float32)
        sc = jnp.dot(oh, scale_ref[:], preferred_element_type=jnp.float32)
        sh = jnp.dot(oh, shift_ref[:], preferred_element_type=jnp.float32)
        eacc = sc * eacc + sh
    eout_ref[:] = eacc


def _run_c(final, x, xi, p, eacc, Wri1_b, bri1_b, Wri2_b, bri2_b, u_b,
           Wmi_b, bmi_b, Wra1_b, bra1_b, Wra2_b, bra2_b,
           Wro1_b, bro1_b, Wro2_b, bro2_b, Woutp, boutp, ml_Z, scalep, shiftp):
    full = lambda *s: pl.BlockSpec(s, lambda i: (0,) * len(s))
    row = lambda *s: pl.BlockSpec((s[0],) + s[1:],
                                  lambda i: (i,) + (0,) * (len(s) - 1))
    return pl.pallas_call(
        functools.partial(_c_kernel, final),
        grid=(N // TN,),
        in_specs=[
            row(TN, F), row(TN, F),
            pl.BlockSpec((NC, TN, F), lambda i: (0, i, 0)),
            row(TN, F),
            full(NRI, F, F), full(NRI, F), full(NRI, F, F), full(NRI, F),
            full(F), full(F, F), full(F),
            full(NRA, F, F), full(NRA, F), full(NRA, F, F), full(NRA, F),
            full(NRO, F, F), full(NRO, F), full(NRO, F, F), full(NRO, F),
            full(F, F), full(F),
            pl.BlockSpec((1, 1, TN), lambda i: (i, 0, 0)),
            full(95, F), full(95, F),
        ],
        out_specs=[row(TN, F), row(TN, F)],
        out_shape=[
            jax.ShapeDtypeStruct((N, F), jnp.float32),
            jax.ShapeDtypeStruct((N, F), jnp.float32),
        ],
    )(x, xi, p, eacc, Wri1_b, bri1_b, Wri2_b, bri2_b, u_b, Wmi_b, bmi_b,
      Wra1_b, bra1_b, Wra2_b, bra2_b, Wro1_b, bro1_b, Wro2_b, bro2_b,
      Woutp, boutp, ml_Z, scalep, shiftp)


def kernel(mlmm_R, ml_idxi, ml_idxj, ml_Z, emb, centers, widths, Wrbf,
           Wi, bi, Wj, bj, Wri1, bri1, Wri2, bri2, u, Wmi, bmi,
           Wra1, bra1, Wra2, bra2, Wro1, bro1, Wro2, bro2, Wout, bout,
           Escale, Eshift, Qscale, Qshift):
    idxi = ml_idxi.astype(jnp.int32)
    idxj = ml_idxj.astype(jnp.int32)
    z32 = ml_Z.astype(jnp.int32)
    z3 = z32.reshape(N // TN, 1, TN)

    cf, ed = _make_geom()(mlmm_R.reshape(N * 3), idxi, idxj)

    ii2 = idxi.reshape(E // C, C)
    jj2 = idxj.reshape(E // C, C)

    Woutp = jnp.zeros((NB, F, F), jnp.float32).at[:, :, :2].set(Wout)
    boutp = jnp.zeros((NB, F), jnp.float32).at[:, :2].set(bout)
    scalep = jnp.zeros((95, F), jnp.float32).at[:, 0].set(Escale)
    scalep = scalep.at[:, 1].set(Qscale)
    shiftp = jnp.zeros((95, F), jnp.float32).at[:, 0].set(Eshift)
    shiftp = shiftp.at[:, 1].set(Qshift)

    eacc = jnp.zeros((N, F), jnp.float32)
    agg = _make_agg()
    x = None
    for b in range(NB):
        if b == 0:
            x, xi, xj = _run_b0(z3, emb, Wi[0], bi[0], Wj[0], bj[0])
        else:
            xi, xj = _run_b(x, Wi[b], bi[b], Wj[b], bj[b])
        g = _run_g(cf, ed, centers, widths, Wrbf[b])
        p = agg(xj, g, ii2, jj2)
        x, eacc = _run_c(b == NB - 1, x, xi, p, eacc,
                         Wri1[b], bri1[b], Wri2[b], bri2[b], u[b],
                         Wmi[b], bmi[b], Wra1[b], bra1[b], Wra2[b], bra2[b],
                         Wro1[b], bro1[b], Wro2[b], bro2[b],
                         Woutp[b], boutp[b], z3, scalep, shiftp)
    return eacc[:, :2]
